# baseline (speedup 1.0000x reference)
import functools

import jax
import jax.numpy as jnp
from jax import lax
from jax.experimental import pallas as pl
from jax.experimental.pallas import tpu as pltpu
from jax.experimental.pallas import tpu_sc as plsc

NUM_NODES = 100000
EMBED_DIM = 64
BATCH = 16384
COMB_LEN = 20

NC = 2
NS = 16
NW = NC * NS
B_PER_W = BATCH // NW
CB = 32
NCHUNK = B_PER_W // CB
ROWS_PER_CHUNK = CB * COMB_LEN
NGATHER = ROWS_PER_CHUNK // 128

TBLK = 512
TGRID = -(-(NUM_NODES + 1) // TBLK)
WROWS = TGRID * TBLK


def _transpose_body(wt_ref, out_ref):
    xt = wt_ref[...].T
    out_ref[...] = jnp.concatenate([xt, xt], axis=1)


def _sc_body(comb_hbm, weight_hbm, out_hbm, idx_v, cmp_v, rows_v, scr_v,
             outc_v, isem, gsem0, gsem1, osem):
    wid = lax.axis_index("s") * NC + lax.axis_index("c")
    gsems = (gsem0, gsem1)
    lane = lax.iota(jnp.int32, 16)

    for l in range(COMB_LEN):
        pltpu.async_copy(
            comb_hbm.at[l, pl.ds(wid * B_PER_W, B_PER_W)], idx_v.at[l], isem
        )
    for l in range(COMB_LEN):
        pltpu.make_async_copy(
            comb_hbm.at[l, pl.ds(0, B_PER_W)], idx_v.at[l], isem
        ).wait()

    def compact(buf, c):
        for l in range(COMB_LEN):
            v0 = idx_v[l, pl.ds(c * CB, 16)]
            v1 = idx_v[l, pl.ds(c * CB + 16, 16)]
            cmp_v[pl.ds(buf * ROWS_PER_CHUNK + l * CB, 16)] = v0 + v0
            cmp_v[pl.ds(buf * ROWS_PER_CHUNK + l * CB + 16, 16)] = v1 + v1

    def fire_rows(buf):
        for j in range(NGATHER):
            pltpu.async_copy(
                weight_hbm.at[cmp_v.at[pl.ds(buf * ROWS_PER_CHUNK + j * 128,
                                             128)]],
                rows_v.at[buf, pl.ds(j * 128, 128)],
                gsems[buf],
            )

    def drain_rows(buf):
        pltpu.make_async_copy(
            weight_hbm.at[pl.ds(0, ROWS_PER_CHUNK)], rows_v.at[buf], gsems[buf]
        ).wait()

    def drain_out(buf):
        pltpu.make_async_copy(
            outc_v.at[pl.ds(buf * CB, CB)], out_hbm.at[pl.ds(0, CB)], osem
        ).wait()

    def compute(buf, c, need_drain):
        drain_rows(buf)

        def prod_body(e, _):
            acc = [rows_v[buf, e, pl.ds(k * 16, 16)] for k in range(4)]
            for l in range(1, COMB_LEN):
                for k in range(4):
                    acc[k] = acc[k] * rows_v[buf, l * CB + e, pl.ds(k * 16, 16)]
            s = (acc[0] + acc[1]) + (acc[2] + acc[3])
            scr_v[pl.ds(8 + e * 16, 16)] = s
            return ()

        lax.fori_loop(0, CB, prod_body, ())

        for s in (8, 4, 2, 1):
            def round_body(e, _, s=s):
                base = 8 + e * 16
                a = scr_v[pl.ds(base, 16)]
                b = scr_v[pl.ds(base + (s - 2 * (e & s)), 16)]
                scr_v[pl.ds(base, 16)] = a + b
                return ()

            lax.fori_loop(0, CB, round_body, ())

        @pl.when(need_drain)
        def _():
            drain_out(buf)

        for g in range(CB // 16):
            def merge_body(e, res, g=g):
                v = scr_v[pl.ds(8 + (g * 16 + e) * 16, 16)]
                return jnp.where(lane == e, v, res)

            res = lax.fori_loop(0, 16, merge_body,
                                jnp.zeros((16,), jnp.float32))
            outc_v[pl.ds(buf * CB + g * 16, 16)] = res

        pltpu.async_copy(
            outc_v.at[pl.ds(buf * CB, CB)],
            out_hbm.at[pl.ds(wid * B_PER_W + c * CB, CB)],
            osem,
        )

    compact(0, 0)
    fire_rows(0)

    def pair_body(i, _):
        c0 = i * 2
        compact(1, c0 + 1)
        fire_rows(1)
        compute(0, c0, i > 0)

        @pl.when(i < NCHUNK // 2 - 1)
        def _():
            compact(0, c0 + 2)
            fire_rows(0)

        compute(1, c0 + 1, i > 0)
        return ()

    lax.fori_loop(0, NCHUNK // 2, pair_body, ())
    drain_out(0)
    drain_out(1)


@jax.jit
def _hyper_embed(comb_t, weight_t):
    table = pl.pallas_call(
        _transpose_body,
        grid=(TGRID,),
        in_specs=[pl.BlockSpec((EMBED_DIM, TBLK), lambda j: (0, j))],
        out_specs=pl.BlockSpec((TBLK, 128), lambda j: (j, 0)),
        out_shape=jax.ShapeDtypeStruct((WROWS, 128), jnp.float32),
    )(weight_t)

    mesh = plsc.VectorSubcoreMesh(core_axis_name="c", subcore_axis_name="s")
    sc = functools.partial(
        pl.kernel,
        mesh=mesh,
        compiler_params=pltpu.CompilerParams(use_tc_tiling_on_sc=False),
        out_type=jax.ShapeDtypeStruct((BATCH,), jnp.float32),
        scratch_types=[
            pltpu.VMEM((COMB_LEN, B_PER_W), jnp.int32),
            pltpu.VMEM((2 * ROWS_PER_CHUNK,), jnp.int32),
            pltpu.VMEM((2, ROWS_PER_CHUNK, EMBED_DIM), jnp.float32),
            pltpu.VMEM((8 + CB * 16 + 16,), jnp.float32),
            pltpu.VMEM((2 * CB,), jnp.float32),
            pltpu.SemaphoreType.DMA,
            pltpu.SemaphoreType.DMA,
            pltpu.SemaphoreType.DMA,
            pltpu.SemaphoreType.DMA,
        ],
    )(_sc_body)
    return sc(comb_t, table.reshape(2 * WROWS, EMBED_DIM))


def kernel(combinations, weight):
    return _hyper_embed(combinations.astype(jnp.int32).T, weight.T)


# speedup vs baseline: 1.6231x; 1.6231x over previous
import functools

import jax
import jax.numpy as jnp
from jax import lax
from jax.experimental import pallas as pl
from jax.experimental.pallas import tpu as pltpu
from jax.experimental.pallas import tpu_sc as plsc

NUM_NODES = 100000
EMBED_DIM = 64
BATCH = 16384
COMB_LEN = 20

NC = 2
NS = 16
NW = NC * NS
B_PER_W = BATCH // NW
CB = 32
NCHUNK = B_PER_W // CB
ROWS_PER_CHUNK = CB * COMB_LEN
NGATHER = ROWS_PER_CHUNK // 128

TBLK = 2048
TGRID = -(-(NUM_NODES + 1) // TBLK)
WROWS = TGRID * TBLK


def _transpose_body(wt_ref, out_ref):
    xt = wt_ref[...].T
    out_ref[...] = jnp.concatenate([xt, xt], axis=1)


def _sc_body(comb_hbm, weight_hbm, out_hbm, idx_v, cmp_v, rows_v, scr_v,
             outc_v, isem, gsem0, gsem1, osem):
    wid = lax.axis_index("s") * NC + lax.axis_index("c")
    gsems = (gsem0, gsem1)
    lane = lax.iota(jnp.int32, 16)

    for l in range(COMB_LEN):
        pltpu.async_copy(
            comb_hbm.at[l, pl.ds(wid * B_PER_W, B_PER_W)], idx_v.at[l], isem
        )
    for l in range(COMB_LEN):
        pltpu.make_async_copy(
            comb_hbm.at[l, pl.ds(0, B_PER_W)], idx_v.at[l], isem
        ).wait()

    def compact(buf, c):
        for l in range(COMB_LEN):
            v0 = idx_v[l, pl.ds(c * CB, 16)]
            v1 = idx_v[l, pl.ds(c * CB + 16, 16)]
            cmp_v[pl.ds(buf * ROWS_PER_CHUNK + l * CB, 16)] = v0 + v0
            cmp_v[pl.ds(buf * ROWS_PER_CHUNK + l * CB + 16, 16)] = v1 + v1

    def fire_rows(buf):
        for j in range(NGATHER):
            pltpu.async_copy(
                weight_hbm.at[cmp_v.at[pl.ds(buf * ROWS_PER_CHUNK + j * 128,
                                             128)]],
                rows_v.at[buf, pl.ds(j * 128, 128)],
                gsems[buf],
            )

    def drain_rows(buf):
        pltpu.make_async_copy(
            weight_hbm.at[pl.ds(0, ROWS_PER_CHUNK)], rows_v.at[buf], gsems[buf]
        ).wait()

    def drain_out(buf):
        pltpu.make_async_copy(
            outc_v.at[pl.ds(buf * CB, CB)], out_hbm.at[pl.ds(0, CB)], osem
        ).wait()

    def compute(buf, c, need_drain):
        drain_rows(buf)

        def prod_body(e, _):
            acc = [rows_v[buf, e, pl.ds(k * 16, 16)] for k in range(4)]
            for l in range(1, COMB_LEN):
                for k in range(4):
                    acc[k] = acc[k] * rows_v[buf, l * CB + e, pl.ds(k * 16, 16)]
            s = (acc[0] + acc[1]) + (acc[2] + acc[3])
            scr_v[pl.ds(8 + e * 16, 16)] = s
            return ()

        lax.fori_loop(0, CB, prod_body, ())

        for s in (8, 4, 2, 1):
            def round_body(e, _, s=s):
                base = 8 + e * 16
                a = scr_v[pl.ds(base, 16)]
                b = scr_v[pl.ds(base + (s - 2 * (e & s)), 16)]
                scr_v[pl.ds(base, 16)] = a + b
                return ()

            lax.fori_loop(0, CB, round_body, ())

        @pl.when(need_drain)
        def _():
            drain_out(buf)

        for g in range(CB // 16):
            def merge_body(e, res, g=g):
                v = scr_v[pl.ds(8 + (g * 16 + e) * 16, 16)]
                return jnp.where(lane == e, v, res)

            res = lax.fori_loop(0, 16, merge_body,
                                jnp.zeros((16,), jnp.float32))
            outc_v[pl.ds(buf * CB + g * 16, 16)] = res

        pltpu.async_copy(
            outc_v.at[pl.ds(buf * CB, CB)],
            out_hbm.at[pl.ds(wid * B_PER_W + c * CB, CB)],
            osem,
        )

    compact(0, 0)
    fire_rows(0)

    def pair_body(i, _):
        c0 = i * 2
        compact(1, c0 + 1)
        fire_rows(1)
        compute(0, c0, i > 0)

        @pl.when(i < NCHUNK // 2 - 1)
        def _():
            compact(0, c0 + 2)
            fire_rows(0)

        compute(1, c0 + 1, i > 0)
        return ()

    lax.fori_loop(0, NCHUNK // 2, pair_body, ())
    drain_out(0)
    drain_out(1)


@jax.jit
def _hyper_embed(comb_t, weight_t):
    table = pl.pallas_call(
        _transpose_body,
        grid=(TGRID,),
        in_specs=[pl.BlockSpec((EMBED_DIM, TBLK), lambda j: (0, j))],
        out_specs=pl.BlockSpec((TBLK, 128), lambda j: (j, 0)),
        out_shape=jax.ShapeDtypeStruct((WROWS, 128), jnp.float32),
    )(weight_t)

    mesh = plsc.VectorSubcoreMesh(core_axis_name="c", subcore_axis_name="s")
    sc = functools.partial(
        pl.kernel,
        mesh=mesh,
        compiler_params=pltpu.CompilerParams(use_tc_tiling_on_sc=False),
        out_type=jax.ShapeDtypeStruct((BATCH,), jnp.float32),
        scratch_types=[
            pltpu.VMEM((COMB_LEN, B_PER_W), jnp.int32),
            pltpu.VMEM((2 * ROWS_PER_CHUNK,), jnp.int32),
            pltpu.VMEM((2, ROWS_PER_CHUNK, EMBED_DIM), jnp.float32),
            pltpu.VMEM((8 + CB * 16 + 16,), jnp.float32),
            pltpu.VMEM((2 * CB,), jnp.float32),
            pltpu.SemaphoreType.DMA,
            pltpu.SemaphoreType.DMA,
            pltpu.SemaphoreType.DMA,
            pltpu.SemaphoreType.DMA,
        ],
    )(_sc_body)
    return sc(comb_t, table.reshape(2 * WROWS, EMBED_DIM))


def kernel(combinations, weight):
    return _hyper_embed(combinations.astype(jnp.int32).T, weight.T)


# speedup vs baseline: 1.9431x; 1.1972x over previous
import functools

import jax
import jax.numpy as jnp
from jax import lax
from jax.experimental import pallas as pl
from jax.experimental.pallas import tpu as pltpu
from jax.experimental.pallas import tpu_sc as plsc

NUM_NODES = 100000
EMBED_DIM = 64
BATCH = 16384
COMB_LEN = 20

NC = 2
NS = 16
NW = NC * NS
B_PER_W = BATCH // NW
CB = 32
NCHUNK = B_PER_W // CB
ROWS_PER_CHUNK = CB * COMB_LEN
NGATHER = ROWS_PER_CHUNK // 128
NBUF = 4
WPN = EMBED_DIM // 2

TBLK = 4096
TGRID = -(-(NUM_NODES + 1) // TBLK)
WROWS = TGRID * TBLK


QROWS = TBLK // 4


def _transpose_body(wt_ref, out_ref):
    xt = wt_ref[...].T
    xb = jax.lax.bitcast_convert_type(xt, jnp.int32)
    b = (xb + 32767 + ((xb >> 16) & 1)) >> 16
    w = b[:, :WPN] | (b[:, WPN:] << 16)
    out_ref[...] = jnp.concatenate(
        [w[q * QROWS:(q + 1) * QROWS] for q in range(4)], axis=1)


def _sc_body(comb_hbm, weight_hbm, out_hbm, idx_v, cmp_v, rows_v, scr_v,
             outc_v, isem, gsem0, gsem1, gsem2, gsem3, osem):
    wid = lax.axis_index("s") * NC + lax.axis_index("c")
    gsems = (gsem0, gsem1, gsem2, gsem3)
    lane = lax.iota(jnp.int32, 16)

    for l in range(COMB_LEN):
        pltpu.async_copy(
            comb_hbm.at[l, pl.ds(wid * B_PER_W, B_PER_W)], idx_v.at[l], isem
        )
    for l in range(COMB_LEN):
        pltpu.make_async_copy(
            comb_hbm.at[l, pl.ds(0, B_PER_W)], idx_v.at[l], isem
        ).wait()

    def vrow(n):
        return ((n & jnp.int32(-4096)) + ((n & jnp.int32(1023)) << 2)
                + ((n & jnp.int32(4095)) >> 10))

    def compact(buf, c):
        for l in range(COMB_LEN):
            v0 = idx_v[l, pl.ds(c * CB, 16)]
            v1 = idx_v[l, pl.ds(c * CB + 16, 16)]
            cmp_v[pl.ds(buf * ROWS_PER_CHUNK + l * CB, 16)] = vrow(v0)
            cmp_v[pl.ds(buf * ROWS_PER_CHUNK + l * CB + 16, 16)] = vrow(v1)

    def fire_rows(buf):
        for j in range(NGATHER):
            pltpu.async_copy(
                weight_hbm.at[cmp_v.at[pl.ds(buf * ROWS_PER_CHUNK + j * 128,
                                             128)]],
                rows_v.at[buf, pl.ds(j * 128, 128)],
                gsems[buf],
            )

    def drain_rows(buf):
        pltpu.make_async_copy(
            weight_hbm.at[pl.ds(0, ROWS_PER_CHUNK)], rows_v.at[buf], gsems[buf]
        ).wait()

    def drain_out(buf01):
        pltpu.make_async_copy(
            outc_v.at[pl.ds(buf01 * CB, CB)], out_hbm.at[pl.ds(0, CB)], osem
        ).wait()

    def compute(buf, c, need_drain):
        drain_rows(buf)

        mask = jnp.int32(-65536)

        def unpack(w):
            return (plsc.bitcast(w << 16, jnp.float32),
                    plsc.bitcast(w & mask, jnp.float32))

        def prod_body(e, _):
            al0, ah0 = unpack(rows_v[buf, e, pl.ds(0, 16)])
            al1, ah1 = unpack(rows_v[buf, e, pl.ds(16, 16)])
            for l in range(1, COMB_LEN):
                r = l * CB + e
                l0, h0 = unpack(rows_v[buf, r, pl.ds(0, 16)])
                l1, h1 = unpack(rows_v[buf, r, pl.ds(16, 16)])
                al0, ah0 = al0 * l0, ah0 * h0
                al1, ah1 = al1 * l1, ah1 * h1
            scr_v[pl.ds(8 + e * 16, 16)] = (al0 + ah0) + (al1 + ah1)
            return ()

        lax.fori_loop(0, CB, prod_body, ())

        for s in (8, 4, 2, 1):
            def round_body(e, _, s=s):
                base = 8 + e * 16
                a = scr_v[pl.ds(base, 16)]
                b = scr_v[pl.ds(base + (s - 2 * (e & s)), 16)]
                scr_v[pl.ds(base, 16)] = a + b
                return ()

            lax.fori_loop(0, CB, round_body, ())

        @pl.when(need_drain)
        def _():
            drain_out(buf % 2)

        for g in range(CB // 16):
            def merge_body(e, res, g=g):
                v = scr_v[pl.ds(8 + (g * 16 + e) * 16, 16)]
                return jnp.where(lane == e, v, res)

            res = lax.fori_loop(0, 16, merge_body,
                                jnp.zeros((16,), jnp.float32))
            outc_v[pl.ds((buf % 2) * CB + g * 16, 16)] = res

        pltpu.async_copy(
            outc_v.at[pl.ds((buf % 2) * CB, CB)],
            out_hbm.at[pl.ds(wid * B_PER_W + c * CB, CB)],
            osem,
        )

    for c in range(NBUF - 1):
        compact(c, c)
        fire_rows(c)

    def quad_body(i, _):
        c0 = i * NBUF
        for q in range(NBUF):
            c = c0 + q
            bq = (q + NBUF - 1) % NBUF

            @pl.when(c + NBUF - 1 < NCHUNK)
            def _(bq=bq, c=c):
                compact(bq, c + NBUF - 1)
                fire_rows(bq)

            compute(q, c, jnp.logical_or(i > 0, q >= 2))
        return ()

    lax.fori_loop(0, NCHUNK // NBUF, quad_body, ())
    drain_out(0)
    drain_out(1)


@jax.jit
def _hyper_embed(comb_t, weight_t):
    table = pl.pallas_call(
        _transpose_body,
        grid=(TGRID,),
        in_specs=[pl.BlockSpec((EMBED_DIM, TBLK), lambda j: (0, j))],
        out_specs=pl.BlockSpec((TBLK * WPN // 128, 128), lambda j: (j, 0)),
        out_shape=jax.ShapeDtypeStruct((WROWS * WPN // 128, 128), jnp.int32),
    )(weight_t)

    mesh = plsc.VectorSubcoreMesh(core_axis_name="c", subcore_axis_name="s")
    sc = functools.partial(
        pl.kernel,
        mesh=mesh,
        compiler_params=pltpu.CompilerParams(use_tc_tiling_on_sc=False,
                                             needs_layout_passes=False),
        out_type=jax.ShapeDtypeStruct((BATCH,), jnp.float32),
        scratch_types=[
            pltpu.VMEM((COMB_LEN, B_PER_W), jnp.int32),
            pltpu.VMEM((NBUF * ROWS_PER_CHUNK,), jnp.int32),
            pltpu.VMEM((NBUF, ROWS_PER_CHUNK, WPN), jnp.int32),
            pltpu.VMEM((8 + CB * 16 + 16,), jnp.float32),
            pltpu.VMEM((2 * CB,), jnp.float32),
            pltpu.SemaphoreType.DMA,
            pltpu.SemaphoreType.DMA,
            pltpu.SemaphoreType.DMA,
            pltpu.SemaphoreType.DMA,
            pltpu.SemaphoreType.DMA,
            pltpu.SemaphoreType.DMA,
        ],
    )(_sc_body)
    return sc(comb_t, table.reshape(WROWS, WPN))


def kernel(combinations, weight):
    return _hyper_embed(combinations.astype(jnp.int32).T, weight.T)


# speedup vs baseline: 2.0001x; 1.0293x over previous
import functools

import jax
import jax.numpy as jnp
from jax import lax
from jax.experimental import pallas as pl
from jax.experimental.pallas import tpu as pltpu
from jax.experimental.pallas import tpu_sc as plsc

NUM_NODES = 100000
EMBED_DIM = 64
BATCH = 16384
COMB_LEN = 20

NC = 2
NS = 16
NW = NC * NS
B_PER_W = BATCH // NW
CB = 32
NCHUNK = B_PER_W // CB
ROWS_PER_CHUNK = CB * COMB_LEN
NGATHER = ROWS_PER_CHUNK // 128
NBUF = 4
WPN = EMBED_DIM // 2

TBLK = 4096
TGRID = -(-(NUM_NODES + 1) // TBLK)
WROWS = TGRID * TBLK


QROWS = TBLK // 4


def _transpose_body(wt_ref, out_ref):
    x16 = wt_ref[...].astype(jnp.bfloat16)
    eye = (lax.broadcasted_iota(jnp.int32, (EMBED_DIM, EMBED_DIM), 0)
           == lax.broadcasted_iota(jnp.int32, (EMBED_DIM, EMBED_DIM), 1)
           ).astype(jnp.bfloat16)
    xt = lax.dot_general(x16, eye, (((0,), (0,)), ((), ())),
                         preferred_element_type=jnp.float32)
    b = jax.lax.bitcast_convert_type(xt, jnp.int32) >> 16
    w = b[:, :WPN] | (b[:, WPN:] << 16)
    out_ref[...] = jnp.concatenate(
        [w[q * QROWS:(q + 1) * QROWS] for q in range(4)], axis=1)


def _sc_body(comb_hbm, weight_hbm, out_hbm, idx_v, cmp_v, rows_v, scr_v,
             outc_v, isem, gsem0, gsem1, gsem2, gsem3, osem):
    wid = lax.axis_index("s") * NC + lax.axis_index("c")
    gsems = (gsem0, gsem1, gsem2, gsem3)
    lane = lax.iota(jnp.int32, 16)

    for l in range(COMB_LEN):
        pltpu.async_copy(
            comb_hbm.at[l, pl.ds(wid * B_PER_W, B_PER_W)], idx_v.at[l], isem
        )
    for l in range(COMB_LEN):
        pltpu.make_async_copy(
            comb_hbm.at[l, pl.ds(0, B_PER_W)], idx_v.at[l], isem
        ).wait()

    def vrow(n):
        return ((n & jnp.int32(-4096)) + ((n & jnp.int32(1023)) << 2)
                + ((n & jnp.int32(4095)) >> 10))

    def compact(buf, c):
        for l in range(COMB_LEN):
            v0 = idx_v[l, pl.ds(c * CB, 16)]
            v1 = idx_v[l, pl.ds(c * CB + 16, 16)]
            cmp_v[pl.ds(buf * ROWS_PER_CHUNK + l * CB, 16)] = vrow(v0)
            cmp_v[pl.ds(buf * ROWS_PER_CHUNK + l * CB + 16, 16)] = vrow(v1)

    def fire_rows(buf):
        for j in range(NGATHER):
            pltpu.async_copy(
                weight_hbm.at[cmp_v.at[pl.ds(buf * ROWS_PER_CHUNK + j * 128,
                                             128)]],
                rows_v.at[buf, pl.ds(j * 128, 128)],
                gsems[buf],
            )

    def drain_rows(buf):
        pltpu.make_async_copy(
            weight_hbm.at[pl.ds(0, ROWS_PER_CHUNK)], rows_v.at[buf], gsems[buf]
        ).wait()

    def drain_out(buf01):
        pltpu.make_async_copy(
            outc_v.at[pl.ds(buf01 * CB, CB)], out_hbm.at[pl.ds(0, CB)], osem
        ).wait()

    def compute(buf, c, need_drain):
        drain_rows(buf)

        mask = jnp.int32(-65536)

        def unpack(w):
            return (plsc.bitcast(w << 16, jnp.float32),
                    plsc.bitcast(w & mask, jnp.float32))

        def prod_body(e, _):
            al0, ah0 = unpack(rows_v[buf, e, pl.ds(0, 16)])
            al1, ah1 = unpack(rows_v[buf, e, pl.ds(16, 16)])
            for l in range(1, COMB_LEN):
                r = l * CB + e
                l0, h0 = unpack(rows_v[buf, r, pl.ds(0, 16)])
                l1, h1 = unpack(rows_v[buf, r, pl.ds(16, 16)])
                al0, ah0 = al0 * l0, ah0 * h0
                al1, ah1 = al1 * l1, ah1 * h1
            scr_v[pl.ds(8 + e * 16, 16)] = (al0 + ah0) + (al1 + ah1)
            return ()

        lax.fori_loop(0, CB, prod_body, ())

        for s in (8, 4, 2, 1):
            def round_body(e, _, s=s):
                base = 8 + e * 16
                a = scr_v[pl.ds(base, 16)]
                b = scr_v[pl.ds(base + (s - 2 * (e & s)), 16)]
                scr_v[pl.ds(base, 16)] = a + b
                return ()

            lax.fori_loop(0, CB, round_body, ())

        @pl.when(need_drain)
        def _():
            drain_out(buf % 2)

        for g in range(CB // 16):
            def merge_body(e, res, g=g):
                v = scr_v[pl.ds(8 + (g * 16 + e) * 16, 16)]
                return jnp.where(lane == e, v, res)

            res = lax.fori_loop(0, 16, merge_body,
                                jnp.zeros((16,), jnp.float32))
            outc_v[pl.ds((buf % 2) * CB + g * 16, 16)] = res

        pltpu.async_copy(
            outc_v.at[pl.ds((buf % 2) * CB, CB)],
            out_hbm.at[pl.ds(wid * B_PER_W + c * CB, CB)],
            osem,
        )

    for c in range(NBUF - 1):
        compact(c, c)
        fire_rows(c)

    def quad_body(i, _):
        c0 = i * NBUF
        for q in range(NBUF):
            c = c0 + q
            bq = (q + NBUF - 1) % NBUF

            @pl.when(c + NBUF - 1 < NCHUNK)
            def _(bq=bq, c=c):
                compact(bq, c + NBUF - 1)
                fire_rows(bq)

            compute(q, c, jnp.logical_or(i > 0, q >= 2))
        return ()

    lax.fori_loop(0, NCHUNK // NBUF, quad_body, ())
    drain_out(0)
    drain_out(1)


@jax.jit
def _hyper_embed(comb_t, weight_t):
    table = pl.pallas_call(
        _transpose_body,
        grid=(TGRID,),
        in_specs=[pl.BlockSpec((EMBED_DIM, TBLK), lambda j: (0, j))],
        out_specs=pl.BlockSpec((TBLK * WPN // 128, 128), lambda j: (j, 0)),
        out_shape=jax.ShapeDtypeStruct((WROWS * WPN // 128, 128), jnp.int32),
    )(weight_t)

    mesh = plsc.VectorSubcoreMesh(core_axis_name="c", subcore_axis_name="s")
    sc = functools.partial(
        pl.kernel,
        mesh=mesh,
        compiler_params=pltpu.CompilerParams(use_tc_tiling_on_sc=False,
                                             needs_layout_passes=False),
        out_type=jax.ShapeDtypeStruct((BATCH,), jnp.float32),
        scratch_types=[
            pltpu.VMEM((COMB_LEN, B_PER_W), jnp.int32),
            pltpu.VMEM((NBUF * ROWS_PER_CHUNK,), jnp.int32),
            pltpu.VMEM((NBUF, ROWS_PER_CHUNK, WPN), jnp.int32),
            pltpu.VMEM((8 + CB * 16 + 16,), jnp.float32),
            pltpu.VMEM((2 * CB,), jnp.float32),
            pltpu.SemaphoreType.DMA,
            pltpu.SemaphoreType.DMA,
            pltpu.SemaphoreType.DMA,
            pltpu.SemaphoreType.DMA,
            pltpu.SemaphoreType.DMA,
            pltpu.SemaphoreType.DMA,
        ],
    )(_sc_body)
    return sc(comb_t, table.reshape(WROWS, WPN))


def kernel(combinations, weight):
    return _hyper_embed(combinations.astype(jnp.int32).T, weight.T)


# speedup vs baseline: 2.1180x; 1.0590x over previous
import functools

import jax
import jax.numpy as jnp
from jax import lax
from jax.experimental import pallas as pl
from jax.experimental.pallas import tpu as pltpu
from jax.experimental.pallas import tpu_sc as plsc

NUM_NODES = 100000
EMBED_DIM = 64
BATCH = 16384
COMB_LEN = 20

NC = 2
NS = 16
NW = NC * NS
B_PER_W = BATCH // NW
CB = 32
NCHUNK = B_PER_W // CB
ROWS_PER_CHUNK = CB * COMB_LEN
NGATHER = ROWS_PER_CHUNK // 128
NBUF = 4
WPN = EMBED_DIM // 2

TBLK = 8192
TGRID = -(-(NUM_NODES + 1) // TBLK)
WROWS = TGRID * TBLK


QROWS = TBLK // 4


def _transpose_body(wt_ref, out_ref):
    x16 = wt_ref[...].astype(jnp.bfloat16)
    eye = (lax.broadcasted_iota(jnp.int32, (EMBED_DIM, EMBED_DIM), 0)
           == lax.broadcasted_iota(jnp.int32, (EMBED_DIM, EMBED_DIM), 1)
           ).astype(jnp.bfloat16)
    xt = lax.dot_general(x16, eye, (((0,), (0,)), ((), ())),
                         preferred_element_type=jnp.float32)
    b = jax.lax.bitcast_convert_type(xt, jnp.int32) >> 16
    w = b[:, :WPN] | (b[:, WPN:] << 16)
    out_ref[...] = jnp.concatenate(
        [w[q * QROWS:(q + 1) * QROWS] for q in range(4)], axis=1)


def _sc_body(comb_hbm, weight_hbm, out_hbm, idx_v, cmp_v, rows_v, scr_v,
             outc_v, isem, gsem0, gsem1, gsem2, gsem3, osem):
    wid = lax.axis_index("s") * NC + lax.axis_index("c")
    gsems = (gsem0, gsem1, gsem2, gsem3)
    lane = lax.iota(jnp.int32, 16)

    for l in range(COMB_LEN):
        pltpu.async_copy(
            comb_hbm.at[l, pl.ds(wid * B_PER_W, B_PER_W)], idx_v.at[l], isem
        )
    for l in range(COMB_LEN):
        pltpu.make_async_copy(
            comb_hbm.at[l, pl.ds(0, B_PER_W)], idx_v.at[l], isem
        ).wait()

    def vrow(n):
        return ((n & jnp.int32(-TBLK)) + ((n & jnp.int32(QROWS - 1)) << 2)
                + ((n & jnp.int32(TBLK - 1)) >> (TBLK // 4).bit_length() - 1))

    def compact(buf, c):
        for l in range(COMB_LEN):
            v0 = idx_v[l, pl.ds(c * CB, 16)]
            v1 = idx_v[l, pl.ds(c * CB + 16, 16)]
            cmp_v[pl.ds(buf * ROWS_PER_CHUNK + l * CB, 16)] = vrow(v0)
            cmp_v[pl.ds(buf * ROWS_PER_CHUNK + l * CB + 16, 16)] = vrow(v1)

    def fire_rows(buf):
        for j in range(NGATHER):
            pltpu.async_copy(
                weight_hbm.at[cmp_v.at[pl.ds(buf * ROWS_PER_CHUNK + j * 128,
                                             128)]],
                rows_v.at[buf, pl.ds(j * 128, 128)],
                gsems[buf],
            )

    def drain_rows(buf):
        pltpu.make_async_copy(
            weight_hbm.at[pl.ds(0, ROWS_PER_CHUNK)], rows_v.at[buf], gsems[buf]
        ).wait()

    def drain_out(buf01):
        pltpu.make_async_copy(
            outc_v.at[pl.ds(buf01 * CB, CB)], out_hbm.at[pl.ds(0, CB)], osem
        ).wait()

    def compute(buf, c, need_drain):
        drain_rows(buf)

        mask = jnp.int32(-65536)

        def unpack(w):
            return (plsc.bitcast(w << 16, jnp.float32),
                    plsc.bitcast(w & mask, jnp.float32))

        def prod_body(e, _):
            al0, ah0 = unpack(rows_v[buf, e, pl.ds(0, 16)])
            al1, ah1 = unpack(rows_v[buf, e, pl.ds(16, 16)])
            for l in range(1, COMB_LEN):
                r = l * CB + e
                l0, h0 = unpack(rows_v[buf, r, pl.ds(0, 16)])
                l1, h1 = unpack(rows_v[buf, r, pl.ds(16, 16)])
                al0, ah0 = al0 * l0, ah0 * h0
                al1, ah1 = al1 * l1, ah1 * h1
            scr_v[pl.ds(8 + e * 16, 16)] = (al0 + ah0) + (al1 + ah1)
            return ()

        lax.fori_loop(0, CB, prod_body, ())

        for s in (8, 4, 2, 1):
            def round_body(e, _, s=s):
                base = 8 + e * 16
                a = scr_v[pl.ds(base, 16)]
                b = scr_v[pl.ds(base + (s - 2 * (e & s)), 16)]
                scr_v[pl.ds(base, 16)] = a + b
                return ()

            lax.fori_loop(0, CB, round_body, ())

        @pl.when(need_drain)
        def _():
            drain_out(buf % 2)

        for g in range(CB // 16):
            def merge_body(e, res, g=g):
                v = scr_v[pl.ds(8 + (g * 16 + e) * 16, 16)]
                return jnp.where(lane == e, v, res)

            res = lax.fori_loop(0, 16, merge_body,
                                jnp.zeros((16,), jnp.float32))
            outc_v[pl.ds((buf % 2) * CB + g * 16, 16)] = res

        pltpu.async_copy(
            outc_v.at[pl.ds((buf % 2) * CB, CB)],
            out_hbm.at[pl.ds(wid * B_PER_W + c * CB, CB)],
            osem,
        )

    for c in range(NBUF - 1):
        compact(c, c)
        fire_rows(c)

    def quad_body(i, _):
        c0 = i * NBUF
        for q in range(NBUF):
            c = c0 + q
            bq = (q + NBUF - 1) % NBUF

            @pl.when(c + NBUF - 1 < NCHUNK)
            def _(bq=bq, c=c):
                compact(bq, c + NBUF - 1)
                fire_rows(bq)

            compute(q, c, jnp.logical_or(i > 0, q >= 2))
        return ()

    lax.fori_loop(0, NCHUNK // NBUF, quad_body, ())
    drain_out(0)
    drain_out(1)


@jax.jit
def _hyper_embed(comb_t, weight_t):
    table = pl.pallas_call(
        _transpose_body,
        grid=(TGRID,),
        in_specs=[pl.BlockSpec((EMBED_DIM, TBLK), lambda j: (0, j))],
        out_specs=pl.BlockSpec((TBLK * WPN // 128, 128), lambda j: (j, 0)),
        out_shape=jax.ShapeDtypeStruct((WROWS * WPN // 128, 128), jnp.int32),
    )(weight_t)

    mesh = plsc.VectorSubcoreMesh(core_axis_name="c", subcore_axis_name="s")
    sc = functools.partial(
        pl.kernel,
        mesh=mesh,
        compiler_params=pltpu.CompilerParams(use_tc_tiling_on_sc=False,
                                             needs_layout_passes=False),
        out_type=jax.ShapeDtypeStruct((BATCH,), jnp.float32),
        scratch_types=[
            pltpu.VMEM((COMB_LEN, B_PER_W), jnp.int32),
            pltpu.VMEM((NBUF * ROWS_PER_CHUNK,), jnp.int32),
            pltpu.VMEM((NBUF, ROWS_PER_CHUNK, WPN), jnp.int32),
            pltpu.VMEM((8 + CB * 16 + 16,), jnp.float32),
            pltpu.VMEM((2 * CB,), jnp.float32),
            pltpu.SemaphoreType.DMA,
            pltpu.SemaphoreType.DMA,
            pltpu.SemaphoreType.DMA,
            pltpu.SemaphoreType.DMA,
            pltpu.SemaphoreType.DMA,
            pltpu.SemaphoreType.DMA,
        ],
    )(_sc_body)
    return sc(comb_t, table.reshape(WROWS, WPN))


def kernel(combinations, weight):
    return _hyper_embed(combinations.astype(jnp.int32).T, weight.T)
